# baseline (speedup 1.0000x reference)
import functools

import jax
import jax.numpy as jnp
from jax import lax
from jax.experimental import pallas as pl
from jax.experimental.pallas import tpu as pltpu
from jax.experimental.pallas import tpu_sc as plsc

N_BLK = 1000

NR = 4
NF = 8
FS = 16
BLK = 1000
BLKP = 1008
CQ = 1024
GC = 64
NGC = CQ // GC
UNROLL = 5


def _mm_pre_body(z_ref, w_ref, a_ref, b_ref):
    p = jnp.dot(z_ref[...], w_ref[...], preferred_element_type=jnp.float32)
    h = a_ref.shape[1]
    a_ref[...] = p[:, :h]
    b_ref[...] = p[:, h:]


def _mm_post_body(z_ref, mmax_ref, a_ref, wu_ref, bu_ref, bm_ref, out_ref):
    mmax = mmax_ref[...]
    m = jnp.where(jnp.isneginf(mmax), 0.0, mmax + a_ref[...] + bm_ref[...])
    inp = jnp.concatenate([z_ref[...], m], axis=1)
    out_ref[...] = jnp.dot(inp, wu_ref[...],
                           preferred_element_type=jnp.float32) + bu_ref[...]


def _sc_body(n, e, dests_hbm, srcs_hbm, w_hbm, b8_hbm, wcol_hbm, m_hbm,
             d_stage, s_stage, w_stage, cq_gi, cq_d, cq_w, rows, m_a, m_b,
             wcol_v, sem, sem_stage):
    rn = n // NR
    nb = e // BLK
    cid = lax.axis_index("c")
    sid = lax.axis_index("s")
    wid = sid * 2 + cid
    r = wid % NR
    fc = wid // NR
    lo = r * rn

    pltpu.sync_copy(wcol_hbm, wcol_v)
    wc = wcol_v[pl.ds(fc * FS, FS)]

    iota = lax.iota(jnp.int32, 16)
    neginf = jnp.full((16,), -jnp.inf, jnp.float32)
    zero16 = jnp.zeros((16,), jnp.int32)

    def init_m(i, carry):
        m_a[i, :] = neginf
        m_b[i, :] = neginf
        return carry

    lax.fori_loop(0, rn + 1, init_m, 0)

    def init_q(i, carry):
        cq_gi[pl.ds(i * 16, 16)] = zero16
        return carry

    lax.fori_loop(0, CQ // 16, init_q, 0)

    def stage_start(b):
        buf = b & 1
        off = b * BLK
        pltpu.async_copy(dests_hbm.at[pl.ds(off, BLK)],
                         d_stage.at[buf, pl.ds(0, BLK)], sem_stage)
        pltpu.async_copy(srcs_hbm.at[pl.ds(off, BLK)],
                         s_stage.at[buf, pl.ds(0, BLK)], sem_stage)
        pltpu.async_copy(w_hbm.at[pl.ds(off, BLK)],
                         w_stage.at[buf, pl.ds(0, BLK)], sem_stage)

    def stage_wait(b):
        buf = b & 1
        off = b * BLK
        pltpu.make_async_copy(dests_hbm.at[pl.ds(off, BLK)],
                              d_stage.at[buf, pl.ds(0, BLK)],
                              sem_stage).wait()
        pltpu.make_async_copy(srcs_hbm.at[pl.ds(off, BLK)],
                              s_stage.at[buf, pl.ds(0, BLK)],
                              sem_stage).wait()
        pltpu.make_async_copy(w_hbm.at[pl.ds(off, BLK)],
                              w_stage.at[buf, pl.ds(0, BLK)],
                              sem_stage).wait()

    def rmw_block(buf, cnt):
        def rmw(p, carry):
            for q in range(2):
                e0 = 4 * p + 2 * q
                e1 = e0 + 1
                ev0 = jnp.full((16,), e0, jnp.int32)
                ev1 = jnp.full((16,), e1, jnp.int32)
                d0 = plsc.load_gather(cq_d.at[buf], [ev0])
                d1 = plsc.load_gather(cq_d.at[buf], [ev1])
                w0 = plsc.load_gather(cq_w.at[buf], [ev0])
                w1 = plsc.load_gather(cq_w.at[buf], [ev1])
                s0 = rows[buf, e0, :] + w0 * wc
                s1 = rows[buf, e1, :] + w1 * wc
                c0 = plsc.load_gather(m_a, [d0, iota])
                c1 = plsc.load_gather(m_b, [d1, iota])
                plsc.store_scatter(m_a, [d0, iota], jnp.maximum(c0, s0))
                plsc.store_scatter(m_b, [d1, iota], jnp.maximum(c1, s1))
            return carry

        lax.fori_loop(0, cnt >> 2, rmw, 0)

    stage_start(0)

    def block(b, cnt_prev):
        buf = b & 1
        prv = 1 - buf
        with jax.named_scope("sc_stage_wait"):
            stage_wait(b)

            @pl.when(b + 1 < nb)
            def _():
                stage_start(b + 1)

        def filt(k, cnt_vec, lane_valid=None):
            dv = d_stage[buf, pl.ds(k * 16, 16)]
            dl = dv - lo
            mask = (dl >= 0) & (dl < rn)
            if lane_valid is not None:
                mask = mask & lane_valid
            csum = plsc.cumsum(jnp.where(mask, 1, 0))
            pos = cnt_vec + csum - 1
            sv = s_stage[buf, pl.ds(k * 16, 16)]
            plsc.store_scatter(cq_gi, [pos], sv + fc, mask=mask)
            plsc.store_scatter(cq_d.at[buf], [pos], dl, mask=mask)
            plsc.store_scatter(cq_w.at[buf], [pos],
                               w_stage[buf, pl.ds(k * 16, 16)], mask=mask)
            return cnt_vec + plsc.all_reduce_population_count(mask)

        def filtu(k0, cnt_vec):
            for u in range(UNROLL):
                cnt_vec = filt(k0 * UNROLL + u, cnt_vec)
            return cnt_vec

        with jax.named_scope("sc_filter"):
            nk = BLK // 16
            rem = BLK - nk * 16
            cnt_vec = lax.fori_loop(0, nk // UNROLL, filtu,
                                    jnp.zeros((16,), jnp.int32))
            for k in range((nk // UNROLL) * UNROLL, nk):
                cnt_vec = filt(k, cnt_vec)
            if rem:
                cnt_vec = filt(nk, cnt_vec, lane_valid=iota < rem)
            cnt = jnp.max(cnt_vec)

            padn = (-cnt) & 3

            @pl.when(padn > 0)
            def _():
                pos = jnp.full((16,), cnt, jnp.int32) + iota
                pmask = iota < padn
                plsc.store_scatter(cq_d.at[buf], [pos],
                                   jnp.full((16,), rn, jnp.int32),
                                   mask=pmask)
                plsc.store_scatter(cq_w.at[buf], [pos],
                                   jnp.zeros((16,), jnp.float32),
                                   mask=pmask)
                plsc.store_scatter(cq_gi, [pos], zero16, mask=pmask)

            cnt = cnt + padn

        with jax.named_scope("sc_gather_start"):
            for c in range(NGC):
                @pl.when(c * GC < cnt)
                def _():
                    pltpu.async_copy(b8_hbm.at[cq_gi.at[pl.ds(c * GC, GC)]],
                                     rows.at[buf, pl.ds(c * GC, GC), :], sem)

        with jax.named_scope("sc_rmw"):
            rmw_block(prv, cnt_prev)

        with jax.named_scope("sc_gather_wait"):
            for c in range(NGC):
                @pl.when(c * GC < cnt)
                def _():
                    pltpu.make_async_copy(
                        b8_hbm.at[cq_gi.at[pl.ds(c * GC, GC)]],
                        rows.at[buf, pl.ds(c * GC, GC), :], sem).wait()

        return cnt

    cnt_last = lax.fori_loop(0, nb, block, jnp.int32(0))
    with jax.named_scope("sc_rmw_last"):
        rmw_block((nb - 1) & 1, cnt_last)

    def merge(i, carry):
        m_a[i, :] = jnp.maximum(m_a[i, :], m_b[i, :])
        return carry

    lax.fori_loop(0, rn, merge, 0)

    pltpu.sync_copy(m_a.at[pl.ds(0, rn), :],
                    m_hbm.at[pl.ds(lo, rn), pl.ds(fc * FS, FS)])


def _scatter_max_sc(n, e, dests, srcs, w_flat, b8, wcol):
    mesh = plsc.VectorSubcoreMesh(core_axis_name="c", subcore_axis_name="s")
    return pl.kernel(
        functools.partial(_sc_body, n, e),
        out_type=jax.ShapeDtypeStruct((n, NF * FS), jnp.float32),
        mesh=mesh,
        scratch_types=[
            pltpu.VMEM((2, BLKP), jnp.int32),
            pltpu.VMEM((2, BLKP), jnp.int32),
            pltpu.VMEM((2, BLKP), jnp.float32),
            pltpu.VMEM((CQ,), jnp.int32),
            pltpu.VMEM((2, CQ), jnp.int32),
            pltpu.VMEM((2, CQ), jnp.float32),
            pltpu.VMEM((2, CQ, FS), jnp.float32),
            pltpu.VMEM((n // NR + 1, FS), jnp.float32),
            pltpu.VMEM((n // NR + 1, FS), jnp.float32),
            pltpu.VMEM((NF * FS,), jnp.float32),
            pltpu.SemaphoreType.DMA,
            pltpu.SemaphoreType.DMA,
        ],
        compiler_params=pltpu.CompilerParams(needs_layout_passes=False,
                                             use_tc_tiling_on_sc=False),
    )(dests, srcs, w_flat, b8, wcol)


def kernel(sources, dests, weights, z, Wm, bm, Wu, bu):
    n, h = z.shape
    e = sources.shape[0]
    sources = sources.astype(jnp.int32)
    dests = dests.astype(jnp.int32)
    wcol = Wm[:, 2 * h]

    grid = n // N_BLK
    A, B = pl.pallas_call(
        _mm_pre_body,
        grid=(grid,),
        in_specs=[
            pl.BlockSpec((N_BLK, h), lambda i: (i, 0)),
            pl.BlockSpec((h, 2 * h), lambda i: (0, 0)),
        ],
        out_specs=[
            pl.BlockSpec((N_BLK, h), lambda i: (i, 0)),
            pl.BlockSpec((N_BLK, h), lambda i: (i, 0)),
        ],
        out_shape=[
            jax.ShapeDtypeStruct((n, h), jnp.float32),
            jax.ShapeDtypeStruct((n, h), jnp.float32),
        ],
    )(z, jnp.concatenate([Wm[:, :h].T, Wm[:, h:2 * h].T], axis=1))

    b8 = B.reshape(n * NF, FS)
    mmax = _scatter_max_sc(n, e, dests, sources * NF, weights.reshape(e),
                           b8, wcol)

    out = pl.pallas_call(
        _mm_post_body,
        grid=(grid,),
        in_specs=[
            pl.BlockSpec((N_BLK, h), lambda i: (i, 0)),
            pl.BlockSpec((N_BLK, h), lambda i: (i, 0)),
            pl.BlockSpec((N_BLK, h), lambda i: (i, 0)),
            pl.BlockSpec((2 * h, h), lambda i: (0, 0)),
            pl.BlockSpec((1, h), lambda i: (0, 0)),
            pl.BlockSpec((1, h), lambda i: (0, 0)),
        ],
        out_specs=pl.BlockSpec((N_BLK, h), lambda i: (i, 0)),
        out_shape=jax.ShapeDtypeStruct((n, h), jnp.float32),
    )(z, mmax, A, Wu.T, bu[None, :], bm[None, :])
    return out


# speedup vs baseline: 1.0106x; 1.0106x over previous
import functools

import jax
import jax.numpy as jnp
from jax import lax
from jax.experimental import pallas as pl
from jax.experimental.pallas import tpu as pltpu
from jax.experimental.pallas import tpu_sc as plsc

N_BLK = 1000

NR = 4
NF = 8
FS = 16
BLK = 1000
BLKP = 1008
CQ = 1024
GC = 64
NGC = CQ // GC
UNROLL = 5


def _mm_pre_body(z_ref, w_ref, a_ref, b_ref):
    p = jnp.dot(z_ref[...], w_ref[...], preferred_element_type=jnp.float32)
    h = a_ref.shape[1]
    a_ref[...] = p[:, :h]
    b_ref[...] = p[:, h:]


def _mm_post_body(z_ref, mmax_ref, a_ref, wu_ref, bu_ref, bm_ref, out_ref):
    mmax = mmax_ref[...]
    m = jnp.where(jnp.isneginf(mmax), 0.0, mmax + a_ref[...] + bm_ref[...])
    inp = jnp.concatenate([z_ref[...], m], axis=1)
    out_ref[...] = jnp.dot(inp, wu_ref[...],
                           preferred_element_type=jnp.float32) + bu_ref[...]


def _sc_body(n, e, dests_hbm, srcs_hbm, w_hbm, b8_hbm, wcol_hbm, m_hbm,
             d_stage, s_stage, w_stage, cq_gi, cq_d, cq_w, rows, m_a, m_b,
             wcol_v, sem, sem_stage):
    rn = n // NR
    nb = e // BLK
    cid = lax.axis_index("c")
    sid = lax.axis_index("s")
    wid = sid * 2 + cid
    r = wid % NR
    fc = wid // NR
    lo = r * rn

    pltpu.sync_copy(wcol_hbm, wcol_v)
    wc = wcol_v[pl.ds(fc * FS, FS)]

    iota = lax.iota(jnp.int32, 16)
    neginf = jnp.full((16,), -jnp.inf, jnp.float32)
    zero16 = jnp.zeros((16,), jnp.int32)

    def init_m(i, carry):
        m_a[i, :] = neginf
        m_b[i, :] = neginf
        return carry

    lax.fori_loop(0, rn + 1, init_m, 0)

    def init_q(i, carry):
        cq_gi[pl.ds(i * 16, 16)] = zero16
        return carry

    lax.fori_loop(0, CQ // 16, init_q, 0)

    def stage_start(b):
        buf = b & 1
        off = b * BLK
        pltpu.async_copy(dests_hbm.at[pl.ds(off, BLK)],
                         d_stage.at[buf, pl.ds(0, BLK)], sem_stage)
        pltpu.async_copy(srcs_hbm.at[pl.ds(off, BLK)],
                         s_stage.at[buf, pl.ds(0, BLK)], sem_stage)
        pltpu.async_copy(w_hbm.at[pl.ds(off, BLK)],
                         w_stage.at[buf, pl.ds(0, BLK)], sem_stage)

    def stage_wait(b):
        buf = b & 1
        off = b * BLK
        pltpu.make_async_copy(dests_hbm.at[pl.ds(off, BLK)],
                              d_stage.at[buf, pl.ds(0, BLK)],
                              sem_stage).wait()
        pltpu.make_async_copy(srcs_hbm.at[pl.ds(off, BLK)],
                              s_stage.at[buf, pl.ds(0, BLK)],
                              sem_stage).wait()
        pltpu.make_async_copy(w_hbm.at[pl.ds(off, BLK)],
                              w_stage.at[buf, pl.ds(0, BLK)],
                              sem_stage).wait()

    def rmw_block(buf, cnt):
        def rmw(p, carry):
            e0 = 2 * p
            e1 = 2 * p + 1
            ev0 = jnp.full((16,), e0, jnp.int32)
            ev1 = jnp.full((16,), e1, jnp.int32)
            d0 = plsc.load_gather(cq_d.at[buf], [ev0])
            d1 = plsc.load_gather(cq_d.at[buf], [ev1])
            w0 = plsc.load_gather(cq_w.at[buf], [ev0])
            w1 = plsc.load_gather(cq_w.at[buf], [ev1])
            s0 = rows[buf, e0, :] + w0 * wc
            s1 = rows[buf, e1, :] + w1 * wc
            c0 = plsc.load_gather(m_a, [d0, iota])
            c1 = plsc.load_gather(m_b, [d1, iota])
            plsc.store_scatter(m_a, [d0, iota], jnp.maximum(c0, s0))
            plsc.store_scatter(m_b, [d1, iota], jnp.maximum(c1, s1))
            return carry

        lax.fori_loop(0, cnt >> 1, rmw, 0)

    stage_start(0)

    def block(b, cnt_prev):
        buf = b & 1
        prv = 1 - buf
        with jax.named_scope("sc_stage_wait"):
            stage_wait(b)

            @pl.when(b + 1 < nb)
            def _():
                stage_start(b + 1)

        def filt(k, cnt_vec, lane_valid=None):
            dv = d_stage[buf, pl.ds(k * 16, 16)]
            dl = dv - lo
            mask = (dl >= 0) & (dl < rn)
            if lane_valid is not None:
                mask = mask & lane_valid
            csum = plsc.cumsum(jnp.where(mask, 1, 0))
            pos = cnt_vec + csum - 1
            sv = s_stage[buf, pl.ds(k * 16, 16)]
            plsc.store_scatter(cq_gi, [pos], sv + fc, mask=mask)
            plsc.store_scatter(cq_d.at[buf], [pos], dl, mask=mask)
            plsc.store_scatter(cq_w.at[buf], [pos],
                               w_stage[buf, pl.ds(k * 16, 16)], mask=mask)
            return cnt_vec + plsc.all_reduce_population_count(mask)

        def filtu(k0, cnt_vec):
            for u in range(UNROLL):
                cnt_vec = filt(k0 * UNROLL + u, cnt_vec)
            return cnt_vec

        with jax.named_scope("sc_filter"):
            nk = BLK // 16
            rem = BLK - nk * 16
            cnt_vec = lax.fori_loop(0, nk // UNROLL, filtu,
                                    jnp.zeros((16,), jnp.int32))
            for k in range((nk // UNROLL) * UNROLL, nk):
                cnt_vec = filt(k, cnt_vec)
            if rem:
                cnt_vec = filt(nk, cnt_vec, lane_valid=iota < rem)
            cnt = jnp.max(cnt_vec)

            padn = cnt & 1

            @pl.when(padn > 0)
            def _():
                pos = jnp.full((16,), cnt, jnp.int32) + iota
                pmask = iota < padn
                plsc.store_scatter(cq_d.at[buf], [pos],
                                   jnp.full((16,), rn, jnp.int32),
                                   mask=pmask)
                plsc.store_scatter(cq_w.at[buf], [pos],
                                   jnp.zeros((16,), jnp.float32),
                                   mask=pmask)
                plsc.store_scatter(cq_gi, [pos], zero16, mask=pmask)

            cnt = cnt + padn

        with jax.named_scope("sc_gather_start"):
            for c in range(NGC):
                @pl.when(c * GC < cnt)
                def _():
                    pltpu.async_copy(b8_hbm.at[cq_gi.at[pl.ds(c * GC, GC)]],
                                     rows.at[buf, pl.ds(c * GC, GC), :], sem)

        with jax.named_scope("sc_rmw"):
            rmw_block(prv, cnt_prev)

        with jax.named_scope("sc_gather_wait"):
            for c in range(NGC):
                @pl.when(c * GC < cnt)
                def _():
                    pltpu.make_async_copy(
                        b8_hbm.at[cq_gi.at[pl.ds(c * GC, GC)]],
                        rows.at[buf, pl.ds(c * GC, GC), :], sem).wait()

        return cnt

    cnt_last = lax.fori_loop(0, nb, block, jnp.int32(0))
    with jax.named_scope("sc_rmw_last"):
        rmw_block((nb - 1) & 1, cnt_last)

    def merge(i, carry):
        m_a[i, :] = jnp.maximum(m_a[i, :], m_b[i, :])
        return carry

    lax.fori_loop(0, rn, merge, 0)

    pltpu.sync_copy(m_a.at[pl.ds(0, rn), :],
                    m_hbm.at[pl.ds(lo, rn), pl.ds(fc * FS, FS)])


def _scatter_max_sc(n, e, dests, srcs, w_flat, b8, wcol):
    mesh = plsc.VectorSubcoreMesh(core_axis_name="c", subcore_axis_name="s")
    return pl.kernel(
        functools.partial(_sc_body, n, e),
        out_type=jax.ShapeDtypeStruct((n, NF * FS), jnp.float32),
        mesh=mesh,
        scratch_types=[
            pltpu.VMEM((2, BLKP), jnp.int32),
            pltpu.VMEM((2, BLKP), jnp.int32),
            pltpu.VMEM((2, BLKP), jnp.float32),
            pltpu.VMEM((CQ,), jnp.int32),
            pltpu.VMEM((2, CQ), jnp.int32),
            pltpu.VMEM((2, CQ), jnp.float32),
            pltpu.VMEM((2, CQ, FS), jnp.float32),
            pltpu.VMEM((n // NR + 1, FS), jnp.float32),
            pltpu.VMEM((n // NR + 1, FS), jnp.float32),
            pltpu.VMEM((NF * FS,), jnp.float32),
            pltpu.SemaphoreType.DMA,
            pltpu.SemaphoreType.DMA,
        ],
        compiler_params=pltpu.CompilerParams(needs_layout_passes=False,
                                             use_tc_tiling_on_sc=False),
    )(dests, srcs, w_flat, b8, wcol)


def kernel(sources, dests, weights, z, Wm, bm, Wu, bu):
    n, h = z.shape
    e = sources.shape[0]
    sources = sources.astype(jnp.int32)
    dests = dests.astype(jnp.int32)
    wcol = Wm[:, 2 * h]

    grid = n // N_BLK
    A, B = pl.pallas_call(
        _mm_pre_body,
        grid=(grid,),
        in_specs=[
            pl.BlockSpec((N_BLK, h), lambda i: (i, 0)),
            pl.BlockSpec((h, 2 * h), lambda i: (0, 0)),
        ],
        out_specs=[
            pl.BlockSpec((N_BLK, h), lambda i: (i, 0)),
            pl.BlockSpec((N_BLK, h), lambda i: (i, 0)),
        ],
        out_shape=[
            jax.ShapeDtypeStruct((n, h), jnp.float32),
            jax.ShapeDtypeStruct((n, h), jnp.float32),
        ],
    )(z, jnp.concatenate([Wm[:, :h].T, Wm[:, h:2 * h].T], axis=1))

    b8 = B.reshape(n * NF, FS)
    mmax = _scatter_max_sc(n, e, dests, sources * NF, weights.reshape(e),
                           b8, wcol)

    out = pl.pallas_call(
        _mm_post_body,
        grid=(grid,),
        in_specs=[
            pl.BlockSpec((N_BLK, h), lambda i: (i, 0)),
            pl.BlockSpec((N_BLK, h), lambda i: (i, 0)),
            pl.BlockSpec((N_BLK, h), lambda i: (i, 0)),
            pl.BlockSpec((2 * h, h), lambda i: (0, 0)),
            pl.BlockSpec((1, h), lambda i: (0, 0)),
            pl.BlockSpec((1, h), lambda i: (0, 0)),
        ],
        out_specs=pl.BlockSpec((N_BLK, h), lambda i: (i, 0)),
        out_shape=jax.ShapeDtypeStruct((n, h), jnp.float32),
    )(z, mmax, A, Wu.T, bu[None, :], bm[None, :])
    return out
